# baseline (speedup 1.0000x reference)
import jax
import jax.numpy as jnp
import numpy as np
from jax.experimental import pallas as pl
from jax.experimental.pallas import tpu as pltpu
from jax.sharding import Mesh, PartitionSpec as P

LANE = 128
SUBLANE = 8
MAX_TILE_B = 512


def _round_up(n, m):
    return (n + m - 1) // m * m


def _mlp_body(x_ref, w1_ref, b1_ref, w2_ref, b2_ref, o_ref):
    xb = x_ref[...].astype(jnp.bfloat16)
    w1b = w1_ref[...].astype(jnp.bfloat16)
    h = jnp.dot(xb, w1b, preferred_element_type=jnp.float32)
    h = jnp.maximum(h + b1_ref[...], 0.0).astype(jnp.bfloat16)
    w2b = w2_ref[...].astype(jnp.bfloat16)
    y = jnp.dot(h, w2b, preferred_element_type=jnp.float32)
    o_ref[...] = y + b2_ref[...]


def _forward(x, w1_p, b1_p, w2_p, b2_p):
    B, d_in = x.shape
    d_in_p, h_p = w1_p.shape
    _, d_out_p = w2_p.shape

    tile_b = min(MAX_TILE_B, _round_up(B, SUBLANE))
    b_pad = _round_up(B, tile_b)
    nb = b_pad // tile_b

    if (b_pad, d_in_p) == (B, d_in):
        x_p = x
    else:
        x_p = jnp.zeros((b_pad, d_in_p), x.dtype).at[:B, :d_in].set(x)

    flops = 2 * b_pad * (d_in_p * h_p + h_p * d_out_p)
    bytes_accessed = 4 * (
        b_pad * d_in_p
        + d_in_p * h_p + h_p
        + h_p * d_out_p + d_out_p
        + b_pad * d_out_p
    )

    out_p = pl.pallas_call(
        _mlp_body,
        out_shape=jax.ShapeDtypeStruct((b_pad, d_out_p), jnp.float32),
        grid_spec=pltpu.PrefetchScalarGridSpec(
            num_scalar_prefetch=0,
            grid=(nb,),
            in_specs=[
                pl.BlockSpec((tile_b, d_in_p), lambda i: (i, 0)),
                pl.BlockSpec((d_in_p, h_p), lambda i: (0, 0)),
                pl.BlockSpec((1, h_p), lambda i: (0, 0)),
                pl.BlockSpec((h_p, d_out_p), lambda i: (0, 0)),
                pl.BlockSpec((1, d_out_p), lambda i: (0, 0)),
            ],
            out_specs=pl.BlockSpec((tile_b, d_out_p), lambda i: (i, 0)),
        ),
        compiler_params=pltpu.CompilerParams(
            dimension_semantics=("parallel",),
        ),
        cost_estimate=pl.CostEstimate(
            flops=flops, transcendentals=0, bytes_accessed=bytes_accessed
        ),
    )(x_p, w1_p, b1_p, w2_p, b2_p)

    return out_p[:B, :]


def kernel(x, w1_p, b1_p, w2_p, b2_p):
    B = x.shape[0]
    d_out = 1024
    devs = jax.devices()
    half = B // 2
    if len(devs) >= 2 and B % 2 == 0 and half % SUBLANE == 0:
        mesh = Mesh(np.array(devs[:2]), ("d",))
        fwd = jax.shard_map(
            _forward,
            mesh=mesh,
            in_specs=(P("d", None), P(None, None), P(None, None),
                      P(None, None), P(None, None)),
            out_specs=P("d", None),
            check_vma=False,
        )
        out = fwd(x, w1_p, b1_p, w2_p, b2_p)
    else:
        out = _forward(x, w1_p, b1_p, w2_p, b2_p)
    return out[:, :d_out]


# speedup vs baseline: 3.0934x; 3.0934x over previous
import jax
import jax.numpy as jnp
from jax.experimental import pallas as pl
from jax.experimental.pallas import tpu as pltpu

LANE = 128
SUBLANE = 8
TILE_B = 1024
CHUNK_H = 1024


def _round_up(n, m):
    return (n + m - 1) // m * m


def _mlp_body(x_ref, w1_ref, b1_ref, w2_ref, b2_ref, o_ref, xb_s):
    k = pl.program_id(1)

    @pl.when(k == 0)
    def _cache_x():
        xb_s[...] = x_ref[...].astype(jnp.bfloat16)

    h = jnp.dot(xb_s[...], w1_ref[...].astype(jnp.bfloat16),
                preferred_element_type=jnp.float32)
    h = jnp.maximum(h + b1_ref[...], 0.0).astype(jnp.bfloat16)
    yk = jnp.dot(h, w2_ref[...].astype(jnp.bfloat16),
                 preferred_element_type=jnp.float32)

    @pl.when(k == 0)
    def _init():
        o_ref[...] = yk + b2_ref[...]

    @pl.when(k > 0)
    def _acc():
        o_ref[...] += yk


def _forward(x, w1_p, b1_p, w2_p, b2_p):
    B, d_in = x.shape
    d_in_p, h_p = w1_p.shape
    _, d_out_p = w2_p.shape

    tile_b = min(TILE_B, _round_up(B, SUBLANE))
    b_pad = _round_up(B, tile_b)
    nb = b_pad // tile_b
    chunk_h = min(CHUNK_H, h_p)
    nk = h_p // chunk_h if h_p % chunk_h == 0 else 1
    if nk == 1:
        chunk_h = h_p

    if (b_pad, d_in_p) == (B, d_in):
        x_p = x
    else:
        x_p = jnp.zeros((b_pad, d_in_p), x.dtype).at[:B, :d_in].set(x)

    flops = 2 * b_pad * (d_in_p * h_p + h_p * d_out_p)
    bytes_accessed = 4 * (
        b_pad * d_in_p
        + nb * (d_in_p * h_p + h_p)
        + nb * (h_p * d_out_p + d_out_p)
        + b_pad * d_out_p
    )

    out_p = pl.pallas_call(
        _mlp_body,
        out_shape=jax.ShapeDtypeStruct((b_pad, d_out_p), jnp.float32),
        grid_spec=pltpu.PrefetchScalarGridSpec(
            num_scalar_prefetch=0,
            grid=(nb, nk),
            in_specs=[
                pl.BlockSpec((tile_b, d_in_p), lambda i, k: (i, 0)),
                pl.BlockSpec((d_in_p, chunk_h), lambda i, k: (0, k)),
                pl.BlockSpec((1, chunk_h), lambda i, k: (0, k)),
                pl.BlockSpec((chunk_h, d_out_p), lambda i, k: (k, 0)),
                pl.BlockSpec((1, d_out_p), lambda i, k: (0, 0)),
            ],
            out_specs=pl.BlockSpec((tile_b, d_out_p), lambda i, k: (i, 0)),
            scratch_shapes=[
                pltpu.VMEM((tile_b, d_in_p), jnp.bfloat16),
            ],
        ),
        compiler_params=pltpu.CompilerParams(
            dimension_semantics=("parallel", "arbitrary"),
        ),
        cost_estimate=pl.CostEstimate(
            flops=flops, transcendentals=0, bytes_accessed=bytes_accessed
        ),
    )(x_p, w1_p, b1_p, w2_p, b2_p)

    return out_p[:B, :]


def kernel(x, w1_p, b1_p, w2_p, b2_p):
    d_out = 1024
    return _forward(x, w1_p, b1_p, w2_p, b2_p)[:, :d_out]


# speedup vs baseline: 3.3535x; 1.0841x over previous
import jax
import jax.numpy as jnp
from jax.experimental import pallas as pl
from jax.experimental.pallas import tpu as pltpu

LANE = 128
SUBLANE = 8
TILE_B = 1024
CHUNK_H = 1024


def _round_up(n, m):
    return (n + m - 1) // m * m


def _make_body(n_chunks, chunk_h):
    def _mlp_body(x_ref, w1_ref, b1_ref, w2_ref, b2_ref, o_ref):
        x = x_ref[...]
        y = b2_ref[...]
        for c in range(n_chunks):
            lo = c * chunk_h
            hi = lo + chunk_h
            h = jnp.dot(x, w1_ref[:, lo:hi],
                        preferred_element_type=jnp.float32)
            h = jnp.maximum(h + b1_ref[:, lo:hi], 0.0)
            y = y + jnp.dot(h, w2_ref[lo:hi, :],
                            preferred_element_type=jnp.float32)
        o_ref[...] = y
    return _mlp_body


def _forward(x, w1_p, b1_p, w2_p, b2_p):
    B, d_in = x.shape
    d_in_p, h_p = w1_p.shape
    _, d_out_p = w2_p.shape

    tile_b = min(TILE_B, _round_up(B, SUBLANE))
    b_pad = _round_up(B, tile_b)
    nb = b_pad // tile_b
    if h_p % CHUNK_H == 0:
        chunk_h, n_chunks = CHUNK_H, h_p // CHUNK_H
    else:
        chunk_h, n_chunks = h_p, 1

    if (b_pad, d_in_p) == (B, d_in):
        x_p = x
    else:
        x_p = jnp.zeros((b_pad, d_in_p), x.dtype).at[:B, :d_in].set(x)

    flops = 2 * b_pad * (d_in_p * h_p + h_p * d_out_p)
    bytes_accessed = 4 * (
        b_pad * d_in_p
        + d_in_p * h_p + h_p
        + h_p * d_out_p + d_out_p
        + b_pad * d_out_p
    )

    out_p = pl.pallas_call(
        _make_body(n_chunks, chunk_h),
        out_shape=jax.ShapeDtypeStruct((b_pad, d_out_p), jnp.float32),
        grid_spec=pltpu.PrefetchScalarGridSpec(
            num_scalar_prefetch=0,
            grid=(nb,),
            in_specs=[
                pl.BlockSpec((tile_b, d_in_p), lambda i: (i, 0)),
                pl.BlockSpec((d_in_p, h_p), lambda i: (0, 0)),
                pl.BlockSpec((1, h_p), lambda i: (0, 0)),
                pl.BlockSpec((h_p, d_out_p), lambda i: (0, 0)),
                pl.BlockSpec((1, d_out_p), lambda i: (0, 0)),
            ],
            out_specs=pl.BlockSpec((tile_b, d_out_p), lambda i: (i, 0)),
        ),
        compiler_params=pltpu.CompilerParams(
            dimension_semantics=("parallel",),
        ),
        cost_estimate=pl.CostEstimate(
            flops=flops, transcendentals=0, bytes_accessed=bytes_accessed
        ),
    )(x_p, w1_p, b1_p, w2_p, b2_p)

    return out_p[:B, :]


def kernel(x, w1_p, b1_p, w2_p, b2_p):
    d_out = 1024
    return _forward(x, w1_p, b1_p, w2_p, b2_p)[:, :d_out]
